# baseline (speedup 1.0000x reference)
import functools

import jax
import jax.numpy as jnp
from jax import lax
from jax.experimental import pallas as pl
from jax.experimental.pallas import tpu as pltpu
from jax.experimental.pallas import tpu_sc as plsc

NE = 64
K = 2
BR = 512
N = 16384
NWORK = 32
RPW = N // NWORK
L = 16
NG = 2
GE = NE // NG


def _logits_block(x_ref, w_ref, b_ref, val_ref, idx_ref):
    lt = lax.dot_general(
        w_ref[...], x_ref[...],
        dimension_numbers=(((1,), (1,)), ((), ())),
        preferred_element_type=jnp.float32,
    ) + b_ref[...]
    srow = lax.broadcasted_iota(jnp.int32, (GE, BR), 0)
    v1s, i1s, v2s, i2s = [], [], [], []
    for g in range(NG):
        blk = lt[GE * g:GE * (g + 1), :]
        m1 = jnp.max(blk, axis=0, keepdims=True)
        r1 = jnp.min(jnp.where(blk == m1, srow, GE), axis=0, keepdims=True)
        masked = jnp.where(srow == r1, -jnp.inf, blk)
        m2 = jnp.max(masked, axis=0, keepdims=True)
        r2 = jnp.min(jnp.where(masked == m2, srow, GE), axis=0, keepdims=True)
        v1s.append(m1)
        i1s.append(r1 + GE * g)
        v2s.append(m2)
        i2s.append(r2 + GE * g)
    val_ref[...] = jnp.concatenate(v1s + v2s, axis=0)
    idx_ref[...] = jnp.concatenate(i1s + i2s, axis=0)


def _tc_group_top2(x_flat, W, bcol):
    return pl.pallas_call(
        _logits_block,
        grid=(N // BR,),
        in_specs=[
            pl.BlockSpec((BR, x_flat.shape[1]), lambda i: (i, 0)),
            pl.BlockSpec((NE, x_flat.shape[1]), lambda i: (0, 0)),
            pl.BlockSpec((NE, 1), lambda i: (0, 0)),
        ],
        out_specs=[
            pl.BlockSpec((2 * NG, BR), lambda i: (0, i)),
            pl.BlockSpec((2 * NG, BR), lambda i: (0, i)),
        ],
        out_shape=[
            jax.ShapeDtypeStruct((2 * NG, N), jnp.float32),
            jax.ShapeDtypeStruct((2 * NG, N), jnp.int32),
        ],
        compiler_params=pltpu.CompilerParams(
            dimension_semantics=("parallel",),
        ),
    )(x_flat, W, bcol)


def _merge_top2(a, b):
    am1, ai1, am2, ai2 = a
    bm1, bi1, bm2, bi2 = b
    gt = bm1 > am1
    m1 = jnp.where(gt, bm1, am1)
    i1 = jnp.where(gt, bi1, ai1)
    cv = jnp.where(gt, am1, am2)
    ci = jnp.where(gt, ai1, ai2)
    dv = jnp.where(gt, bm2, bm1)
    di = jnp.where(gt, bi2, bi1)
    gt2 = dv > cv
    m2 = jnp.where(gt2, dv, cv)
    i2 = jnp.where(gt2, di, ci)
    return (m1, i1, m2, i2)


@functools.partial(
    pl.kernel,
    mesh=plsc.VectorSubcoreMesh(core_axis_name="c", subcore_axis_name="s"),
    out_type=[
        jax.ShapeDtypeStruct((K, N), jnp.float32),
        jax.ShapeDtypeStruct((K, N), jnp.int32),
    ],
    scratch_types=[
        pltpu.VMEM((2 * NG, RPW), jnp.float32),
        pltpu.VMEM((2 * NG, RPW), jnp.int32),
        pltpu.VMEM((K, RPW), jnp.float32),
        pltpu.VMEM((K, RPW), jnp.int32),
        pltpu.SemaphoreType.DMA,
        pltpu.SemaphoreType.DMA,
    ],
)
def _sc_top2(val_hbm, idx_hbm, wout_hbm, iout_hbm, vslab, islab, wv, iv, sem0, sem1):
    nc = 2
    wid = lax.axis_index("s") * nc + lax.axis_index("c")
    base = wid * RPW
    cp0 = pltpu.async_copy(val_hbm.at[:, pl.ds(base, RPW)], vslab, sem0)
    cp1 = pltpu.async_copy(idx_hbm.at[:, pl.ds(base, RPW)], islab, sem1)
    cp0.wait()
    cp1.wait()

    def jbody(j, carry):
        off = j * L
        cur = (
            vslab[0, pl.ds(off, L)],
            islab[0, pl.ds(off, L)],
            vslab[NG, pl.ds(off, L)],
            islab[NG, pl.ds(off, L)],
        )
        for g in range(1, NG):
            nxt = (
                vslab[g, pl.ds(off, L)],
                islab[g, pl.ds(off, L)],
                vslab[NG + g, pl.ds(off, L)],
                islab[NG + g, pl.ds(off, L)],
            )
            cur = _merge_top2(cur, nxt)
        m1, i1, m2, i2 = cur
        e2 = jnp.exp(m2 - m1)
        den = 1.0 + e2
        wv[0, pl.ds(off, L)] = 1.0 / den
        wv[1, pl.ds(off, L)] = e2 / den
        iv[0, pl.ds(off, L)] = i1
        iv[1, pl.ds(off, L)] = i2
        return carry

    lax.fori_loop(0, RPW // L, jbody, 0)
    pltpu.sync_copy(wv, wout_hbm.at[:, pl.ds(base, RPW)])
    pltpu.sync_copy(iv, iout_hbm.at[:, pl.ds(base, RPW)])


@jax.jit
def _router(x_flat, W, bcol):
    vals, idxs = _tc_group_top2(x_flat, W, bcol)
    w_t, i_t = _sc_top2(vals, idxs)
    return w_t, i_t


def kernel(x, W, b, training=False):
    batch, seq, hidden = x.shape
    x_flat = x.reshape(-1, hidden)
    w_t, i_t = _router(x_flat, W, b.reshape(NE, 1))
    top_k_weights = w_t.T.reshape(batch, seq, K)
    expert_indices = i_t.T.reshape(batch, seq, K)
    aux_loss = jnp.float32(0.0)
    return (top_k_weights, expert_indices, aux_loss)


# speedup vs baseline: 1.1212x; 1.1212x over previous
import functools

import jax
import jax.numpy as jnp
from jax import lax
from jax.experimental import pallas as pl
from jax.experimental.pallas import tpu as pltpu
from jax.experimental.pallas import tpu_sc as plsc

NE = 64
K = 2
BR = 1024
N = 16384
NWORK = 32
RPW = N // NWORK
L = 16
NG = 2
GE = NE // NG


def _logits_block(x_ref, w_ref, b_ref, val_ref, idx_ref):
    lt = lax.dot_general(
        w_ref[...], x_ref[...],
        dimension_numbers=(((1,), (1,)), ((), ())),
        preferred_element_type=jnp.float32,
    ) + b_ref[...]
    srow = lax.broadcasted_iota(jnp.int32, (GE, BR), 0)
    v1s, i1s, v2s, i2s = [], [], [], []
    for g in range(NG):
        blk = lt[GE * g:GE * (g + 1), :]
        m1 = jnp.max(blk, axis=0, keepdims=True)
        r1 = jnp.min(jnp.where(blk == m1, srow, GE), axis=0, keepdims=True)
        masked = jnp.where(srow == r1, -jnp.inf, blk)
        m2 = jnp.max(masked, axis=0, keepdims=True)
        r2 = jnp.min(jnp.where(masked == m2, srow, GE), axis=0, keepdims=True)
        v1s.append(m1)
        i1s.append(r1 + GE * g)
        v2s.append(m2)
        i2s.append(r2 + GE * g)
    val_ref[...] = jnp.concatenate(v1s + v2s, axis=0)
    idx_ref[...] = jnp.concatenate(i1s + i2s, axis=0)


def _tc_group_top2(x_flat, W, bcol):
    return pl.pallas_call(
        _logits_block,
        grid=(N // BR,),
        in_specs=[
            pl.BlockSpec((BR, x_flat.shape[1]), lambda i: (i, 0)),
            pl.BlockSpec((NE, x_flat.shape[1]), lambda i: (0, 0)),
            pl.BlockSpec((NE, 1), lambda i: (0, 0)),
        ],
        out_specs=[
            pl.BlockSpec((2 * NG, BR), lambda i: (0, i)),
            pl.BlockSpec((2 * NG, BR), lambda i: (0, i)),
        ],
        out_shape=[
            jax.ShapeDtypeStruct((2 * NG, N), jnp.float32),
            jax.ShapeDtypeStruct((2 * NG, N), jnp.int32),
        ],
        compiler_params=pltpu.CompilerParams(
            dimension_semantics=("parallel",),
        ),
    )(x_flat, W, bcol)


def _merge_top2(a, b):
    am1, ai1, am2, ai2 = a
    bm1, bi1, bm2, bi2 = b
    gt = bm1 > am1
    m1 = jnp.where(gt, bm1, am1)
    i1 = jnp.where(gt, bi1, ai1)
    cv = jnp.where(gt, am1, am2)
    ci = jnp.where(gt, ai1, ai2)
    dv = jnp.where(gt, bm2, bm1)
    di = jnp.where(gt, bi2, bi1)
    gt2 = dv > cv
    m2 = jnp.where(gt2, dv, cv)
    i2 = jnp.where(gt2, di, ci)
    return (m1, i1, m2, i2)


@functools.partial(
    pl.kernel,
    mesh=plsc.VectorSubcoreMesh(core_axis_name="c", subcore_axis_name="s"),
    out_type=[
        jax.ShapeDtypeStruct((K, N), jnp.float32),
        jax.ShapeDtypeStruct((K, N), jnp.int32),
    ],
    scratch_types=[
        pltpu.VMEM((2 * NG, RPW), jnp.float32),
        pltpu.VMEM((2 * NG, RPW), jnp.int32),
        pltpu.VMEM((K, RPW), jnp.float32),
        pltpu.VMEM((K, RPW), jnp.int32),
        pltpu.SemaphoreType.DMA,
        pltpu.SemaphoreType.DMA,
    ],
)
def _sc_top2(val_hbm, idx_hbm, wout_hbm, iout_hbm, vslab, islab, wv, iv, sem0, sem1):
    nc = 2
    wid = lax.axis_index("s") * nc + lax.axis_index("c")
    base = wid * RPW
    cp0 = pltpu.async_copy(val_hbm.at[:, pl.ds(base, RPW)], vslab, sem0)
    cp1 = pltpu.async_copy(idx_hbm.at[:, pl.ds(base, RPW)], islab, sem1)
    cp0.wait()
    cp1.wait()

    def jbody(j, carry):
        off = j * L
        cur = (
            vslab[0, pl.ds(off, L)],
            islab[0, pl.ds(off, L)],
            vslab[NG, pl.ds(off, L)],
            islab[NG, pl.ds(off, L)],
        )
        for g in range(1, NG):
            nxt = (
                vslab[g, pl.ds(off, L)],
                islab[g, pl.ds(off, L)],
                vslab[NG + g, pl.ds(off, L)],
                islab[NG + g, pl.ds(off, L)],
            )
            cur = _merge_top2(cur, nxt)
        m1, i1, m2, i2 = cur
        e2 = jnp.exp(m2 - m1)
        den = 1.0 + e2
        wv[0, pl.ds(off, L)] = 1.0 / den
        wv[1, pl.ds(off, L)] = e2 / den
        iv[0, pl.ds(off, L)] = i1
        iv[1, pl.ds(off, L)] = i2
        return carry

    lax.fori_loop(0, RPW // L, jbody, 0)
    pltpu.sync_copy(wv, wout_hbm.at[:, pl.ds(base, RPW)])
    pltpu.sync_copy(iv, iout_hbm.at[:, pl.ds(base, RPW)])


@jax.jit
def _router(x_flat, W, bcol):
    vals, idxs = _tc_group_top2(x_flat, W, bcol)
    w_t, i_t = _sc_top2(vals, idxs)
    return w_t, i_t


def kernel(x, W, b, training=False):
    batch, seq, hidden = x.shape
    x_flat = x.reshape(-1, hidden)
    w_t, i_t = _router(x_flat, W, b.reshape(NE, 1))
    top_k_weights = w_t.T.reshape(batch, seq, K)
    expert_indices = i_t.T.reshape(batch, seq, K)
    aux_loss = jnp.float32(0.0)
    return (top_k_weights, expert_indices, aux_loss)
